# Initial kernel scaffold; baseline (speedup 1.0000x reference)
import jax
import jax.numpy as jnp
from jax.experimental import pallas as pl


def kernel(input, output):
    raise NotImplementedError("write your pallas kernel here")


# speedup vs baseline: 1.4299x; 1.4299x over previous
import functools

import jax
import jax.numpy as jnp
from jax.experimental import pallas as pl
from jax.experimental.pallas import tpu as pltpu

_TJ = 256
_BIG = 3.0e38


def _chamfer_kernel(c1t_ref, c2_ref, out_ref, rbf_ref, rval_ref, acc_ref, *, nj):
    b = pl.program_id(0)
    j = pl.program_id(1)

    a1 = c1t_ref[0]
    a2 = c2_ref[0]
    n1 = jnp.sum(a1 * a1, axis=0, keepdims=True)
    n2 = jnp.sum(a2 * a2, axis=1, keepdims=True)

    cross_bf = jax.lax.dot_general(
        a2.astype(jnp.bfloat16), a1.astype(jnp.bfloat16),
        (((1,), (0,)), ((), ())),
        preferred_element_type=jnp.float32,
    )
    cross_x = jax.lax.dot_general(
        a2, a1, (((1,), (0,)), ((), ())),
        preferred_element_type=jnp.float32,
        precision=jax.lax.Precision.HIGHEST,
    )

    s = n2 + n1
    d_bf = s - 2.0 * cross_bf
    d_x = s - 2.0 * cross_x

    m_bf = jnp.min(d_bf, axis=1, keepdims=True)
    val2 = jnp.min(jnp.where(d_bf == m_bf, d_x, _BIG), axis=1)
    part = jnp.sum(val2)

    c_bf = jnp.min(d_bf, axis=0, keepdims=True)
    c_val = jnp.min(jnp.where(d_bf == c_bf, d_x, _BIG), axis=0, keepdims=True)

    @pl.when(j == 0)
    def _():
        rbf_ref[...] = c_bf
        rval_ref[...] = c_val

    @pl.when(j > 0)
    def _():
        upd = c_bf < rbf_ref[...]
        rval_ref[...] = jnp.where(upd, c_val, rval_ref[...])
        rbf_ref[...] = jnp.minimum(c_bf, rbf_ref[...])

    @pl.when((b == 0) & (j == 0))
    def _():
        acc_ref[...] = jnp.zeros_like(acc_ref)

    acc_ref[...] = acc_ref[...] + part

    @pl.when(j == nj - 1)
    def _():
        acc_ref[...] = acc_ref[...] + jnp.sum(rval_ref[...])

    out_ref[...] = acc_ref[0:1, 0:1]


@jax.jit
def _chamfer(inp, outp):
    batch, n1, dim = inp.shape
    n2 = outp.shape[1]
    nj = n2 // _TJ
    c1t = jnp.transpose(inp, (0, 2, 1))
    res = pl.pallas_call(
        functools.partial(_chamfer_kernel, nj=nj),
        grid=(batch, nj),
        in_specs=[
            pl.BlockSpec((1, dim, n1), lambda b, j: (b, 0, 0)),
            pl.BlockSpec((1, _TJ, dim), lambda b, j: (b, j, 0)),
        ],
        out_specs=pl.BlockSpec((1, 1), lambda b, j: (0, 0)),
        out_shape=jax.ShapeDtypeStruct((1, 1), jnp.float32),
        scratch_shapes=[
            pltpu.VMEM((1, n1), jnp.float32),
            pltpu.VMEM((1, n1), jnp.float32),
            pltpu.VMEM((1, 128), jnp.float32),
        ],
    )(c1t, outp)
    return res[0, 0]


def kernel(input, output):
    return _chamfer(input, output)


# speedup vs baseline: 1.4736x; 1.0305x over previous
import functools

import jax
import jax.numpy as jnp
from jax.experimental import pallas as pl
from jax.experimental.pallas import tpu as pltpu

_TJ = 512
_BIG = 3.0e38


def _chamfer_kernel(c1t_ref, c2_ref, out_ref, rbf_ref, rval_ref, acc_ref, *, nj):
    b = pl.program_id(0)
    j = pl.program_id(1)

    a1 = c1t_ref[0]
    a2 = c2_ref[0]
    n1 = jnp.sum(a1 * a1, axis=0, keepdims=True)
    n2 = jnp.sum(a2 * a2, axis=1, keepdims=True)

    cross_bf = jax.lax.dot_general(
        a2.astype(jnp.bfloat16), a1.astype(jnp.bfloat16),
        (((1,), (0,)), ((), ())),
        preferred_element_type=jnp.float32,
    )
    cross_x = jax.lax.dot_general(
        a2, a1, (((1,), (0,)), ((), ())),
        preferred_element_type=jnp.float32,
        precision=jax.lax.Precision.HIGHEST,
    )

    s = n2 + n1
    d_bf = s - 2.0 * cross_bf
    d_x = s - 2.0 * cross_x

    m_bf = jnp.min(d_bf, axis=1, keepdims=True)
    val2 = jnp.min(jnp.where(d_bf == m_bf, d_x, _BIG), axis=1)
    part = jnp.sum(val2)

    c_bf = jnp.min(d_bf, axis=0, keepdims=True)
    c_val = jnp.min(jnp.where(d_bf == c_bf, d_x, _BIG), axis=0, keepdims=True)

    @pl.when(j == 0)
    def _():
        rbf_ref[...] = c_bf
        rval_ref[...] = c_val

    @pl.when(j > 0)
    def _():
        upd = c_bf < rbf_ref[...]
        rval_ref[...] = jnp.where(upd, c_val, rval_ref[...])
        rbf_ref[...] = jnp.minimum(c_bf, rbf_ref[...])

    @pl.when((b == 0) & (j == 0))
    def _():
        acc_ref[...] = jnp.zeros_like(acc_ref)

    acc_ref[...] = acc_ref[...] + part

    @pl.when(j == nj - 1)
    def _():
        acc_ref[...] = acc_ref[...] + jnp.sum(rval_ref[...])

    out_ref[...] = acc_ref[0:1, 0:1]


@jax.jit
def _chamfer(inp, outp):
    batch, n1, dim = inp.shape
    n2 = outp.shape[1]
    nj = n2 // _TJ
    c1t = jnp.transpose(inp, (0, 2, 1))
    res = pl.pallas_call(
        functools.partial(_chamfer_kernel, nj=nj),
        grid=(batch, nj),
        in_specs=[
            pl.BlockSpec((1, dim, n1), lambda b, j: (b, 0, 0)),
            pl.BlockSpec((1, _TJ, dim), lambda b, j: (b, j, 0)),
        ],
        out_specs=pl.BlockSpec((1, 1), lambda b, j: (0, 0)),
        out_shape=jax.ShapeDtypeStruct((1, 1), jnp.float32),
        scratch_shapes=[
            pltpu.VMEM((1, n1), jnp.float32),
            pltpu.VMEM((1, n1), jnp.float32),
            pltpu.VMEM((1, 128), jnp.float32),
        ],
    )(c1t, outp)
    return res[0, 0]


def kernel(input, output):
    return _chamfer(input, output)


# speedup vs baseline: 3.1091x; 2.1099x over previous
import functools

import jax
import jax.numpy as jnp
from jax.experimental import pallas as pl
from jax.experimental.pallas import tpu as pltpu

_TJ = 512
_BIG = 3.0e38


def _chamfer_kernel(c1t_ref, c2_ref, out_ref, rbf_ref, rval_ref, acc_ref, *, nj):
    b = pl.program_id(0)
    j = pl.program_id(1)

    a1 = c1t_ref[0]
    a2 = c2_ref[0]
    n1 = jnp.sum(a1 * a1, axis=0, keepdims=True)
    n2 = jnp.sum(a2 * a2, axis=1, keepdims=True)

    a2h = a2.astype(jnp.bfloat16)
    a1h = a1.astype(jnp.bfloat16)
    a2l = (a2 - a2h.astype(jnp.float32)).astype(jnp.bfloat16)
    a1l = (a1 - a1h.astype(jnp.float32)).astype(jnp.bfloat16)

    cross_bf = jax.lax.dot_general(
        a2h, a1h, (((1,), (0,)), ((), ())),
        preferred_element_type=jnp.float32,
    )

    aug2 = jnp.concatenate([a2h, a2l], axis=1)
    aug1 = jnp.concatenate([a1l, a1h], axis=0)
    corr = jax.lax.dot_general(
        aug2, aug1, (((1,), (0,)), ((), ())),
        preferred_element_type=jnp.float32,
    )

    s = n2 + n1
    d_bf = s - 2.0 * cross_bf
    d_x = d_bf - 2.0 * corr

    m_bf = jnp.min(d_bf, axis=1, keepdims=True)
    val2 = jnp.min(jnp.where(d_bf == m_bf, d_x, _BIG), axis=1)
    part = jnp.sum(val2)

    c_bf = jnp.min(d_bf, axis=0, keepdims=True)
    c_val = jnp.min(jnp.where(d_bf == c_bf, d_x, _BIG), axis=0, keepdims=True)

    @pl.when(j == 0)
    def _():
        rbf_ref[...] = c_bf
        rval_ref[...] = c_val

    @pl.when(j > 0)
    def _():
        upd = c_bf < rbf_ref[...]
        rval_ref[...] = jnp.where(upd, c_val, rval_ref[...])
        rbf_ref[...] = jnp.minimum(c_bf, rbf_ref[...])

    @pl.when((b == 0) & (j == 0))
    def _():
        acc_ref[...] = jnp.zeros_like(acc_ref)

    acc_ref[...] = acc_ref[...] + part

    @pl.when(j == nj - 1)
    def _():
        acc_ref[...] = acc_ref[...] + jnp.sum(rval_ref[...])

    out_ref[...] = acc_ref[0:1, 0:1]


@jax.jit
def _chamfer(inp, outp):
    batch, n1, dim = inp.shape
    n2 = outp.shape[1]
    nj = n2 // _TJ
    c1t = jnp.transpose(inp, (0, 2, 1))
    res = pl.pallas_call(
        functools.partial(_chamfer_kernel, nj=nj),
        grid=(batch, nj),
        in_specs=[
            pl.BlockSpec((1, dim, n1), lambda b, j: (b, 0, 0)),
            pl.BlockSpec((1, _TJ, dim), lambda b, j: (b, j, 0)),
        ],
        out_specs=pl.BlockSpec((1, 1), lambda b, j: (0, 0)),
        out_shape=jax.ShapeDtypeStruct((1, 1), jnp.float32),
        scratch_shapes=[
            pltpu.VMEM((1, n1), jnp.float32),
            pltpu.VMEM((1, n1), jnp.float32),
            pltpu.VMEM((1, 128), jnp.float32),
        ],
    )(c1t, outp)
    return res[0, 0]


def kernel(input, output):
    return _chamfer(input, output)


# speedup vs baseline: 3.4970x; 1.1248x over previous
import functools

import jax
import jax.numpy as jnp
from jax.experimental import pallas as pl
from jax.experimental.pallas import tpu as pltpu

_TJ = 512
_BIG = 3.0e38


def _chamfer_kernel(c1t_ref, c2_ref, out_ref, rbf_ref, rval_ref, acc_ref, *, nj):
    b = pl.program_id(0)
    j = pl.program_id(1)

    a1 = c1t_ref[0]
    a2 = c2_ref[0]
    n1 = jnp.sum(a1 * a1, axis=0, keepdims=True)
    n2 = jnp.sum(a2 * a2, axis=1, keepdims=True)

    a2s = -2.0 * a2
    a2h = a2s.astype(jnp.bfloat16)
    a1h = a1.astype(jnp.bfloat16)
    a2l = (a2s - a2h.astype(jnp.float32)).astype(jnp.bfloat16)
    a1l = (a1 - a1h.astype(jnp.float32)).astype(jnp.bfloat16)

    cross_bf = jax.lax.dot_general(
        a2h, a1h, (((1,), (0,)), ((), ())),
        preferred_element_type=jnp.float32,
    )

    aug2 = jnp.concatenate([a2h, a2l], axis=1)
    aug1 = jnp.concatenate([a1l, a1h], axis=0)
    corr = jax.lax.dot_general(
        aug2, aug1, (((1,), (0,)), ((), ())),
        preferred_element_type=jnp.float32,
    )

    s = n2 + n1
    d_bf = s + cross_bf
    d_x = d_bf + corr

    m_bf = jnp.min(d_bf, axis=1, keepdims=True)
    val2 = jnp.min(jnp.where(d_bf == m_bf, d_x, _BIG), axis=1)
    part = jnp.sum(val2)

    c_bf = jnp.min(d_bf, axis=0, keepdims=True)
    c_val = jnp.min(jnp.where(d_bf == c_bf, d_x, _BIG), axis=0, keepdims=True)

    @pl.when(j == 0)
    def _():
        rbf_ref[...] = c_bf
        rval_ref[...] = c_val

    @pl.when(j > 0)
    def _():
        upd = c_bf < rbf_ref[...]
        rval_ref[...] = jnp.where(upd, c_val, rval_ref[...])
        rbf_ref[...] = jnp.minimum(c_bf, rbf_ref[...])

    @pl.when((b == 0) & (j == 0))
    def _():
        acc_ref[...] = jnp.zeros_like(acc_ref)

    acc_ref[...] = acc_ref[...] + part

    @pl.when(j == nj - 1)
    def _():
        acc_ref[...] = acc_ref[...] + jnp.sum(rval_ref[...])

    out_ref[...] = acc_ref[0:1, 0:1]


@jax.jit
def _chamfer(inp, outp):
    batch, n1, dim = inp.shape
    n2 = outp.shape[1]
    nj = n2 // _TJ
    c1t = jnp.transpose(inp, (0, 2, 1))
    res = pl.pallas_call(
        functools.partial(_chamfer_kernel, nj=nj),
        grid=(batch, nj),
        in_specs=[
            pl.BlockSpec((1, dim, n1), lambda b, j: (b, 0, 0)),
            pl.BlockSpec((1, _TJ, dim), lambda b, j: (b, j, 0)),
        ],
        out_specs=pl.BlockSpec((1, 1), lambda b, j: (0, 0)),
        out_shape=jax.ShapeDtypeStruct((1, 1), jnp.float32),
        scratch_shapes=[
            pltpu.VMEM((1, n1), jnp.float32),
            pltpu.VMEM((1, n1), jnp.float32),
            pltpu.VMEM((1, 128), jnp.float32),
        ],
    )(c1t, outp)
    return res[0, 0]


def kernel(input, output):
    return _chamfer(input, output)
